# baseline (speedup 1.0000x reference)
import functools

import jax
import jax.numpy as jnp
from jax import lax
from jax.experimental import pallas as pl
from jax.experimental.pallas import tpu as pltpu
from jax.experimental.pallas import tpu_sc as plsc

N, D, B = 100000, 128, 1024

GR = 2000


def _gate_body(x_ref, w_ref, b_ref, out_ref):
    xb = x_ref[...]
    z = lax.dot_general(xb, w_ref[...], (((1,), (0,)), ((), ())),
                        preferred_element_type=jnp.float32) + b_ref[0]
    out_ref[...] = xb * jax.nn.sigmoid(z)


def _gate_pass(x, W, b):
    return pl.pallas_call(
        _gate_body,
        grid=(N // GR,),
        in_specs=[
            pl.BlockSpec((GR, D), lambda i: (i, 0)),
            pl.BlockSpec((D, 1), lambda i: (0, 0)),
            pl.BlockSpec((1,), lambda i: (0,)),
        ],
        out_specs=pl.BlockSpec((GR, D), lambda i: (i, 0)),
        out_shape=jax.ShapeDtypeStruct((N, D), jnp.float32),
    )(x, W, b)


NC, NS = 2, 16
NW = NC * NS
CH = 80
NCHUNK = N // CH
CPT = NCHUNK // NW
EXTRA = NCHUNK - CPT * NW
ZR = B // NS


def _sc_body(w_hbm, ids_hbm, part_hbm, rowbuf, idxbuf, acc):
    cid = lax.axis_index("c")
    sid = lax.axis_index("s")
    w = cid * NS + sid
    base = w * CPT + jnp.minimum(w, EXTRA)
    nchunks = CPT + (w < EXTRA).astype(jnp.int32)

    def _zrow(r, carry):
        for j in range(D // 16):
            rowbuf[r, pl.ds(j * 16, 16)] = jnp.zeros((16,), jnp.float32)
        return carry
    lax.fori_loop(0, ZR, _zrow, None)
    pltpu.sync_copy(rowbuf.at[pl.ds(0, ZR)], acc.at[pl.ds(sid * ZR, ZR)])
    plsc.subcore_barrier()

    def _chunk(c, carry):
        chunk = base + c
        pltpu.sync_copy(w_hbm.at[pl.ds(chunk * CH, CH), :], rowbuf)
        pltpu.sync_copy(ids_hbm.at[pl.ds(chunk * CH, CH)], idxbuf)
        pltpu.sync_copy(rowbuf, acc.at[idxbuf], add=True)
        return carry
    lax.fori_loop(0, nchunks, _chunk, None)

    plsc.subcore_barrier()
    pltpu.sync_copy(acc.at[pl.ds(sid * ZR, ZR)],
                    part_hbm.at[cid, pl.ds(sid * ZR, ZR)])


def _sc_scatter(weighted, batch_id):
    mesh = plsc.VectorSubcoreMesh(core_axis_name="c", subcore_axis_name="s",
                                  num_cores=NC, num_subcores=NS)
    f = pl.kernel(
        _sc_body,
        out_type=jax.ShapeDtypeStruct((NC, B, D), jnp.float32),
        mesh=mesh,
        scratch_types=[
            pltpu.VMEM((CH, D), jnp.float32),
            pltpu.VMEM((CH,), jnp.int32),
            pltpu.VMEM_SHARED((B, D), jnp.float32),
        ],
    )
    return f(weighted, batch_id)


def _combine_body(p_ref, out_ref):
    out_ref[...] = p_ref[0] + p_ref[1]


def _combine(parts):
    return pl.pallas_call(
        _combine_body,
        in_specs=[pl.BlockSpec((NC, B, D), lambda: (0, 0, 0))],
        out_specs=pl.BlockSpec((B, D), lambda: (0, 0)),
        out_shape=jax.ShapeDtypeStruct((B, D), jnp.float32),
    )(parts)


def kernel(x, batch_id, batch_size, W, b):
    weighted = _gate_pass(x, W, b)
    parts = _sc_scatter(weighted, batch_id)
    out = _combine(parts)
    return out + jnp.asarray(batch_size - B, dtype=out.dtype)


# speedup vs baseline: 1.3452x; 1.3452x over previous
import functools

import jax
import jax.numpy as jnp
from jax import lax
from jax.experimental import pallas as pl
from jax.experimental.pallas import tpu as pltpu
from jax.experimental.pallas import tpu_sc as plsc

N, D, B = 100000, 128, 1024

GR = 2000


def _gate_body(x_ref, w_ref, b_ref, out_ref):
    xb = x_ref[...]
    z = lax.dot_general(xb, w_ref[...], (((1,), (0,)), ((), ())),
                        preferred_element_type=jnp.float32) + b_ref[0]
    out_ref[...] = xb * jax.nn.sigmoid(z)


def _gate_pass(x, W, b):
    return pl.pallas_call(
        _gate_body,
        grid=(N // GR,),
        in_specs=[
            pl.BlockSpec((GR, D), lambda i: (i, 0)),
            pl.BlockSpec((D, 1), lambda i: (0, 0)),
            pl.BlockSpec((1,), lambda i: (0,)),
        ],
        out_specs=pl.BlockSpec((GR, D), lambda i: (i, 0)),
        out_shape=jax.ShapeDtypeStruct((N, D), jnp.float32),
    )(x, W, b)


NC, NS = 2, 16
NW = NC * NS
CH = 80
NCHUNK = N // CH
CPT = NCHUNK // NW
EXTRA = NCHUNK - CPT * NW
ZR = B // NS


SLOTS = CPT + 1


def _sc_body(w_hbm, ids_hbm, part_hbm, rb0, rb1, ib0, ib1, acc,
             sr0, sr1, si0, si1):
    cid = lax.axis_index("c")
    sid = lax.axis_index("s")
    w = cid * NS + sid
    base = w * CPT + jnp.minimum(w, EXTRA)
    n = CPT + (w < EXTRA).astype(jnp.int32)

    rbufs, ibufs, srs, sis = (rb0, rb1), (ib0, ib1), (sr0, sr1), (si0, si1)

    def issue(s, b):
        @pl.when(s < n)
        def _():
            chunk = base + s
            pltpu.async_copy(w_hbm.at[pl.ds(chunk * CH, CH), :],
                             rbufs[b], srs[b])
            pltpu.async_copy(ids_hbm.at[pl.ds(chunk * CH, CH)],
                             ibufs[b], sis[b])

    def _zrow(r, carry):
        for j in range(D // 16):
            rb0[r, pl.ds(j * 16, 16)] = jnp.zeros((16,), jnp.float32)
        return carry
    lax.fori_loop(0, ZR, _zrow, None)
    pltpu.sync_copy(rb0.at[pl.ds(0, ZR)], acc.at[pl.ds(sid * ZR, ZR)])
    plsc.subcore_barrier()

    issue(0, 0)
    issue(1, 1)

    @pl.loop(0, SLOTS, step=2)
    def _slot(o):
        for b in range(2):
            s = o + b

            @pl.when(s < n)
            def _():
                chunk = base + s
                pltpu.make_async_copy(w_hbm.at[pl.ds(chunk * CH, CH), :],
                                      rbufs[b], srs[b]).wait()
                pltpu.make_async_copy(ids_hbm.at[pl.ds(chunk * CH, CH)],
                                      ibufs[b], sis[b]).wait()
                pltpu.sync_copy(rbufs[b], acc.at[ibufs[b]], add=True)
            issue(s + 2, b)

    plsc.subcore_barrier()
    pltpu.sync_copy(acc.at[pl.ds(sid * ZR, ZR)],
                    part_hbm.at[cid, pl.ds(sid * ZR, ZR)])


def _sc_scatter(weighted, batch_id):
    mesh = plsc.VectorSubcoreMesh(core_axis_name="c", subcore_axis_name="s",
                                  num_cores=NC, num_subcores=NS)
    f = pl.kernel(
        _sc_body,
        out_type=jax.ShapeDtypeStruct((NC, B, D), jnp.float32),
        mesh=mesh,
        scratch_types=[
            pltpu.VMEM((CH, D), jnp.float32),
            pltpu.VMEM((CH, D), jnp.float32),
            pltpu.VMEM((CH,), jnp.int32),
            pltpu.VMEM((CH,), jnp.int32),
            pltpu.VMEM_SHARED((B, D), jnp.float32),
            pltpu.SemaphoreType.DMA,
            pltpu.SemaphoreType.DMA,
            pltpu.SemaphoreType.DMA,
            pltpu.SemaphoreType.DMA,
        ],
    )
    return f(weighted, batch_id)


def _combine_body(p_ref, out_ref):
    out_ref[...] = p_ref[0] + p_ref[1]


def _combine(parts):
    return pl.pallas_call(
        _combine_body,
        in_specs=[pl.BlockSpec((NC, B, D), lambda: (0, 0, 0))],
        out_specs=pl.BlockSpec((B, D), lambda: (0, 0)),
        out_shape=jax.ShapeDtypeStruct((B, D), jnp.float32),
    )(parts)


def kernel(x, batch_id, batch_size, W, b):
    weighted = _gate_pass(x, W, b)
    parts = _sc_scatter(weighted, batch_id)
    out = _combine(parts)
    return out + jnp.asarray(batch_size - B, dtype=out.dtype)
